# Initial kernel scaffold; baseline (speedup 1.0000x reference)
import jax
import jax.numpy as jnp
from jax.experimental import pallas as pl


def kernel(memory_bank, components):
    raise NotImplementedError("write your pallas kernel here")


# speedup vs baseline: 1.5581x; 1.5581x over previous
import jax
import jax.numpy as jnp
from jax.experimental import pallas as pl

_N = 1000
_B = 256
_R = 8
_NB = _N // _R
_NB_COMP = _B // _R


def _enqueue_kernel(comp_ref, mem_ref, out_ref):
    i = pl.program_id(0)

    @pl.when(i < _NB_COMP)
    def _():
        out_ref[...] = comp_ref[...]

    @pl.when(i >= _NB_COMP)
    def _():
        out_ref[...] = mem_ref[...]


def kernel(memory_bank, components):
    comps = jax.lax.stop_gradient(components)
    return pl.pallas_call(
        _enqueue_kernel,
        grid=(_NB,),
        in_specs=[
            pl.BlockSpec((_R, 256, 256), lambda i: (jnp.minimum(i, _NB_COMP - 1), 0, 0)),
            pl.BlockSpec((_R, 256, 256), lambda i: (jnp.maximum(i, _NB_COMP), 0, 0)),
        ],
        out_specs=pl.BlockSpec((_R, 256, 256), lambda i: (i, 0, 0)),
        out_shape=jax.ShapeDtypeStruct((_N, 256, 256), memory_bank.dtype),
    )(comps, memory_bank)
